# Initial kernel scaffold; baseline (speedup 1.0000x reference)
import jax
import jax.numpy as jnp
from jax.experimental import pallas as pl


def kernel(features, segment_ids):
    raise NotImplementedError("write your pallas kernel here")


# speedup vs baseline: 4.3501x; 4.3501x over previous
import functools

import jax
import jax.numpy as jnp
from jax import lax
from jax.experimental import pallas as pl
from jax.experimental.pallas import tpu as pltpu
from jax.experimental.pallas import tpu_sc as plsc

N_NODES = 100000
D = 128
S = 256
B = 128
NW = 32
NFULL = N_NODES // B
TAIL = N_NODES - NFULL * B
TAIL_BASE = NFULL * B
TAIL_WID = 31

_mesh = plsc.VectorSubcoreMesh(core_axis_name="c", subcore_axis_name="s")


@functools.partial(
    pl.kernel,
    out_type=jax.ShapeDtypeStruct((2, S, D), jnp.float32),
    mesh=_mesh,
    scratch_types=[
        pltpu.VMEM((B,), jnp.int32),
        pltpu.VMEM((B, D), jnp.float32),
        pltpu.VMEM((TAIL,), jnp.int32),
        pltpu.VMEM((TAIL, D), jnp.float32),
        pltpu.VMEM((16, D), jnp.float32),
        pltpu.VMEM_SHARED((S, D), jnp.float32),
    ],
)
def _sc_segsum(feat_hbm, ids_hbm, out_hbm, idx_v, rows_v, tidx_v, trows_v,
               stage_v, acc_sh):
    cid = lax.axis_index("c")
    sid = lax.axis_index("s")
    wid = sid * 2 + cid

    zeros16 = jnp.zeros((16,), jnp.float32)
    for r in range(16):
        for c in range(D // 16):
            stage_v[r, pl.ds(c * 16, 16)] = zeros16
    pltpu.sync_copy(stage_v, acc_sh.at[pl.ds(sid * 16, 16)])
    plsc.subcore_barrier()

    nb = (NFULL + NW - 1 - wid) // NW

    def body(i, carry):
        base = (wid + i * NW) * B
        pltpu.sync_copy(ids_hbm.at[pl.ds(base, B)], idx_v)
        pltpu.sync_copy(feat_hbm.at[pl.ds(base, B)], rows_v)
        pltpu.sync_copy(rows_v, acc_sh.at[idx_v], add=True)
        return carry

    lax.fori_loop(0, nb, body, 0)

    @pl.when(wid == TAIL_WID)
    def _():
        pltpu.sync_copy(ids_hbm.at[pl.ds(TAIL_BASE, TAIL)], tidx_v)
        pltpu.sync_copy(feat_hbm.at[pl.ds(TAIL_BASE, TAIL)], trows_v)
        pltpu.sync_copy(trows_v, acc_sh.at[tidx_v], add=True)

    plsc.subcore_barrier()

    pltpu.sync_copy(acc_sh.at[pl.ds(sid * 16, 16)], stage_v)
    pltpu.sync_copy(stage_v, out_hbm.at[cid, pl.ds(sid * 16, 16)])


def _combine_body(p_ref, o_ref):
    o_ref[...] = p_ref[0] + p_ref[1]


def kernel(features, segment_ids):
    ids = segment_ids.astype(jnp.int32)
    partials = _sc_segsum(features, ids)
    return pl.pallas_call(
        _combine_body,
        out_shape=jax.ShapeDtypeStruct((S, D), jnp.float32),
    )(partials)


# speedup vs baseline: 6.8223x; 1.5683x over previous
import functools

import jax
import jax.numpy as jnp
from jax import lax
from jax.experimental import pallas as pl
from jax.experimental.pallas import tpu as pltpu
from jax.experimental.pallas import tpu_sc as plsc

N_NODES = 100000
D = 128
S = 256
B = 128
NW = 32
MAXNB = 25
NB31 = 6
TAIL = 32
TAIL_BASE = N_NODES - TAIL

_mesh = plsc.VectorSubcoreMesh(core_axis_name="c", subcore_axis_name="s")


@functools.partial(
    pl.kernel,
    out_type=jax.ShapeDtypeStruct((2, S, D), jnp.float32),
    mesh=_mesh,
    scratch_types=[
        pltpu.VMEM((B,), jnp.int32),
        pltpu.VMEM((B,), jnp.int32),
        pltpu.VMEM((B, D), jnp.float32),
        pltpu.VMEM((B, D), jnp.float32),
        pltpu.VMEM((TAIL,), jnp.int32),
        pltpu.VMEM((TAIL, D), jnp.float32),
        pltpu.VMEM((16, D), jnp.float32),
        pltpu.VMEM_SHARED((S, D), jnp.float32),
        pltpu.SemaphoreType.DMA,
        pltpu.SemaphoreType.DMA,
        pltpu.SemaphoreType.DMA,
        pltpu.SemaphoreType.DMA,
    ],
)
def _sc_segsum(feat_hbm, ids_hbm, out_hbm, idx0, idx1, rows0, rows1,
               tidx_v, trows_v, stage_v, acc_sh, dsem0, dsem1, ssem0, ssem1):
    cid = lax.axis_index("c")
    sid = lax.axis_index("s")
    wid = sid * 2 + cid

    rows = (rows0, rows1)
    idx = (idx0, idx1)
    dsem = (dsem0, dsem1)
    ssem = (ssem0, ssem1)

    zeros16 = jnp.zeros((16,), jnp.float32)
    for r in range(16):
        for c in range(D // 16):
            stage_v[r, pl.ds(c * 16, 16)] = zeros16
    pltpu.sync_copy(stage_v, acc_sh.at[pl.ds(sid * 16, 16)])
    plsc.subcore_barrier()

    row0 = wid * MAXNB * B

    def guard(j):
        return (wid < NW - 1) | (j < NB31)

    def start(j):
        pltpu.async_copy(ids_hbm.at[pl.ds(row0 + j * B, B)], idx[j % 2],
                         dsem[j % 2])
        pltpu.async_copy(feat_hbm.at[pl.ds(row0 + j * B, B)], rows[j % 2],
                         dsem[j % 2])

    def wait_rows(j):
        pltpu.make_async_copy(ids_hbm.at[pl.ds(row0 + j * B, B)],
                              idx[j % 2], dsem[j % 2]).wait()
        pltpu.make_async_copy(feat_hbm.at[pl.ds(row0 + j * B, B)],
                              rows[j % 2], dsem[j % 2]).wait()

    def scat(j):
        pltpu.async_copy(rows[j % 2], acc_sh.at[idx[j % 2]], ssem[j % 2],
                         add=True)

    def wait_scat(j):
        pltpu.make_async_copy(rows[j % 2], acc_sh.at[idx[j % 2]],
                              ssem[j % 2]).wait()

    def maybe(j, fn):
        if j < NB31:
            fn(j)
        else:
            pl.when(guard(j))(lambda: fn(j))

    maybe(0, start)
    for i in range(MAXNB):
        if i + 1 < MAXNB:
            if i - 1 >= 0:
                maybe(i - 1, wait_scat)
            maybe(i + 1, start)
        maybe(i, wait_rows)
        maybe(i, scat)
    maybe(MAXNB - 2, wait_scat)
    maybe(MAXNB - 1, wait_scat)

    @pl.when(wid == NW - 1)
    def _():
        pltpu.sync_copy(ids_hbm.at[pl.ds(TAIL_BASE, TAIL)], tidx_v)
        pltpu.sync_copy(feat_hbm.at[pl.ds(TAIL_BASE, TAIL)], trows_v)
        pltpu.sync_copy(trows_v, acc_sh.at[tidx_v], add=True)

    plsc.subcore_barrier()

    pltpu.sync_copy(acc_sh.at[pl.ds(sid * 16, 16)], stage_v)
    pltpu.sync_copy(stage_v, out_hbm.at[cid, pl.ds(sid * 16, 16)])


def _combine_body(p_ref, o_ref):
    o_ref[...] = p_ref[0] + p_ref[1]


def kernel(features, segment_ids):
    ids = segment_ids.astype(jnp.int32)
    partials = _sc_segsum(features, ids)
    return pl.pallas_call(
        _combine_body,
        out_shape=jax.ShapeDtypeStruct((S, D), jnp.float32),
    )(partials)
